# baseline (speedup 1.0000x reference)
import functools
import jax
import jax.numpy as jnp
from jax import lax
from jax.experimental import pallas as pl
from jax.experimental.pallas import tpu as pltpu
from jax.experimental.pallas import tpu_sc as plsc

N_ENT = 1000000
N_REL = 1000
DIM = 64
B = 16384

NC = 2
NS = 16
L = 16
NW = NC * NS
BPW = B // NW
R = 2 * B
BLK = 128
NBLK_FULL = 7812
TAIL_BASE = NBLK_FULL * BLK
IT_FULL = 244
NORD = 246
DUMP = R
STAGE_ROWS = R + 128
SCB = 64
NRING = 4


def _k1_body(hidx, tidx, ehT, eh_tail, stage,
             idx_all, hits, horder, col0, col1, col2, col3, stage_b, slotbuf,
             cnt_v, off_v, cur_v,
             sem0, sem1, sem2, sem3, ssem):
    wid = lax.axis_index("s") * NC + lax.axis_index("c")
    lanes = lax.iota(jnp.int32, L)
    zero16 = jnp.full((L,), 0, jnp.int32)

    pltpu.sync_copy(hidx, idx_all.at[pl.ds(0, B)])

    def store1(ref, pos, val):
        plsc.store_scatter(ref, [zero16 + pos], zero16 + val,
                           mask=lanes == 0)

    def read1(ref, pos):
        return ref[pl.ds(pos, L)][0]

    def zcnt(i, carry):
        cnt_v[pl.ds(i * L, L)] = zero16
        return carry

    lax.fori_loop(0, (NORD + L - 1) // L + 1, zcnt, 0, unroll=False)

    def scan_for(slot_base):
        def scan(c, nh):
            e = idx_all[pl.ds(c * L, L)]
            slots = jnp.full((L,), slot_base + c * L, jnp.int32) + lanes
            m = lax.bitwise_and(lax.shift_right_logical(e, 7),
                                jnp.full((L,), NW - 1, jnp.int32)) == wid
            packed = lax.bitwise_or(
                lax.bitwise_or(
                    lax.shift_left(lax.shift_right_logical(e, 12),
                                   jnp.full((L,), 22, jnp.int32)),
                    lax.shift_left(
                        lax.bitwise_and(e, jnp.full((L,), BLK - 1,
                                                    jnp.int32)),
                        jnp.full((L,), 15, jnp.int32))),
                slots)
            plsc.store_compressed(hits.at[pl.ds(nh, L)], packed, mask=m)
            n = plsc.all_reduce_population_count(m)[0]
            return nh + n
        return scan

    nhits = lax.fori_loop(0, B // L, scan_for(0), 0, unroll=4)
    pltpu.sync_copy(tidx, idx_all.at[pl.ds(0, B)])
    nhits = lax.fori_loop(0, B // L, scan_for(B), nhits, unroll=4)

    def count(j, carry):
        o = lax.shift_right_logical(read1(hits, j), 22)
        store1(cnt_v, o, read1(cnt_v, o) + 1)
        return carry

    lax.fori_loop(0, nhits, count, 0, unroll=False)

    def prefix(i, s):
        store1(off_v, i, s)
        store1(cur_v, i, s)
        return s + read1(cnt_v, i)

    total = lax.fori_loop(0, NORD, prefix, 0, unroll=False)
    store1(off_v, NORD, total)

    def place(j, carry):
        v = read1(hits, j)
        o = lax.shift_right_logical(v, 22)
        p = read1(cur_v, o)
        store1(horder, p, v)
        store1(cur_v, o, p + 1)
        return carry

    lax.fori_loop(0, nhits, place, 0, unroll=False)

    def prefill(i, carry):
        slotbuf[pl.ds(i * L, L)] = jnp.full((L,), DUMP, jnp.int32)
        return carry

    lax.fori_loop(0, SCB // L, prefill, 0, unroll=False)

    def process_block(i, col, cnt):
        lo = read1(off_v, i)
        hi = read1(off_v, i + 1)

        def extract(j, cnt3):
            flushing = cnt3 == SCB

            @pl.when(flushing)
            def _():
                pltpu.async_copy(stage_b, stage.at[slotbuf], ssem).wait()
                lax.fori_loop(0, SCB // L, prefill, 0, unroll=False)

            cnt3 = jnp.where(flushing, 0, cnt3)
            v = read1(horder, j)
            slot = lax.bitwise_and(v, (1 << 15) - 1)
            el = lax.bitwise_and(lax.shift_right_logical(v, 15), BLK - 1)
            elv = zero16 + el
            rowv = zero16 + cnt3
            for k in range(DIM // L):
                dv = jnp.full((L,), k * L, jnp.int32) + lanes
                vv = plsc.load_gather(col, [dv, elv])
                plsc.store_scatter(stage_b, [rowv, dv], vv)
            store1(slotbuf, cnt3, slot)
            return cnt3 + 1

        return lax.fori_loop(lo, hi, extract, cnt, unroll=False)

    def pbase(i):
        return jnp.minimum((wid + NW * i) * BLK, (NBLK_FULL - 1) * BLK)

    cols = [col0, col1, col2, col3]
    sems = [sem0, sem1, sem2, sem3]
    for k in range(NRING):
        pltpu.async_copy(ehT.at[:, pl.ds(pbase(k), BLK)], cols[k], sems[k])

    def ring(g, cnt):
        for k in range(NRING):
            i = NRING * g + k
            pltpu.make_async_copy(ehT.at[:, pl.ds(pbase(i), BLK)],
                                  cols[k], sems[k]).wait()
            cnt = process_block(i, cols[k], cnt)
            pltpu.async_copy(ehT.at[:, pl.ds(pbase(i + NRING), BLK)],
                             cols[k], sems[k])
        return cnt

    cnt = lax.fori_loop(0, IT_FULL // NRING, ring, 0, unroll=False)

    for k in range(NRING):
        pltpu.make_async_copy(ehT.at[:, pl.ds(pbase(IT_FULL + k), BLK)],
                              cols[k], sems[k]).wait()

    @pl.when(wid < 4)
    def _():
        pltpu.sync_copy(ehT.at[:, pl.ds((wid + NW * IT_FULL) * BLK, BLK)],
                        col0)

    @pl.when(wid == 4)
    def _():
        pltpu.sync_copy(eh_tail, col0)

    cnt = process_block(IT_FULL, col0, cnt)

    @pl.when(cnt > 0)
    def _():
        pltpu.async_copy(stage_b, stage.at[slotbuf], ssem).wait()


def _k2_body(stage, hidx, tidx, ridx, rvh2, b0, b1, out,
             rel_v, rpair_v, hi_v, ti_v,
             h_r0, t_r0, rv_r0, b0_v0, b1_v0,
             h_r1, t_r1, rv_r1, b0_v1, b1_v1,
             out_v, semA, semB):
    wid = lax.axis_index("s") * NC + lax.axis_index("c")
    base = wid * BPW
    lanes = lax.iota(jnp.int32, L)

    pltpu.sync_copy(ridx.at[pl.ds(base, BPW)], rel_v)
    pltpu.sync_copy(hidx.at[pl.ds(base, BPW)], hi_v)
    pltpu.sync_copy(tidx.at[pl.ds(base, BPW)], ti_v)

    def pairs(i, carry):
        s = pl.ds(i * L, L)
        rpair_v[s] = lax.shift_right_logical(rel_v[s], 1)
        return carry

    lax.fori_loop(0, BPW // L, pairs, 0, unroll=False)

    KB = 128
    NBAT = BPW // KB
    sets = [(h_r0, t_r0, rv_r0, b0_v0, b1_v0, semA),
            (h_r1, t_r1, rv_r1, b0_v1, b1_v1, semB)]

    def fire(bi, st):
        h_r, t_r, rv_r, b0_v, b1_v, sem = st
        s0 = base + bi * KB
        return [
            pltpu.async_copy(stage.at[pl.ds(s0, KB), :], h_r, sem),
            pltpu.async_copy(stage.at[pl.ds(B + s0, KB), :], t_r, sem),
            pltpu.async_copy(rvh2.at[rpair_v.at[pl.ds(bi * KB, KB)]],
                             rv_r, sem),
            pltpu.async_copy(b0.at[hi_v.at[pl.ds(bi * KB, KB)]], b0_v, sem),
            pltpu.async_copy(b1.at[ti_v.at[pl.ds(bi * KB, KB)]], b1_v, sem),
        ]

    def compute(bi, st):
        h_r, t_r, rv_r, b0_v, b1_v, _ = st

        def group(g, carry2):
            req = jnp.full((L,), g * L, jnp.int32) + lanes
            rh = lax.bitwise_and(
                rel_v[pl.ds(bi * KB + g * L, L)],
                jnp.full((L,), 1, jnp.int32)) * DIM
            acc = jnp.zeros((L,), jnp.float32)
            for d in range(DIM):
                col = jnp.full((L,), d, jnp.int32)
                hv = plsc.load_gather(h_r, [req, col])
                tv = plsc.load_gather(t_r, [req, col])
                rv = plsc.load_gather(rv_r, [req, rh + col])
                diff = hv - tv - rv
                acc = acc + diff * diff
            gs = pl.ds(g * L, L)
            out_v[pl.ds(bi * KB + g * L, L)] = b0_v[gs] + b1_v[gs] - acc
            return carry2

        lax.fori_loop(0, KB // L, group, 0, unroll=False)

    ring = len(sets)
    pending = [fire(bi, sets[bi]) for bi in range(ring)]
    for bi in range(NBAT):
        st = sets[bi % ring]
        for c in pending[bi % ring]:
            c.wait()
        compute(bi, st)
        if bi + ring < NBAT:
            pending[bi % ring] = fire(bi + ring, st)

    pltpu.sync_copy(out_v, out.at[pl.ds(base, BPW)])


@functools.partial(jax.jit, static_argnames=())
def kernel(head_idx, rel1_idx, tail_idx, rel2_idx, Eh, rvh, bias0, bias1):
    del rel2_idx
    hidx = head_idx.astype(jnp.int32)
    tidx = tail_idx.astype(jnp.int32)
    ridx = rel1_idx.astype(jnp.int32)
    ehT = Eh.T
    eh_tail = jnp.pad(Eh[TAIL_BASE:, :].T, ((0, 0), (0, DIM)))
    rvh2 = rvh.reshape(N_REL // 2, 2 * DIM)
    mesh = plsc.VectorSubcoreMesh(core_axis_name="c", subcore_axis_name="s")

    k1 = pl.kernel(
        _k1_body,
        out_type=jax.ShapeDtypeStruct((STAGE_ROWS, 2 * DIM), jnp.float32),
        mesh=mesh,
        scratch_types=[
            pltpu.VMEM((B + L,), jnp.int32),
            pltpu.VMEM((R + L,), jnp.int32),
            pltpu.VMEM((R + L,), jnp.int32),
            pltpu.VMEM((DIM, BLK), jnp.float32),
            pltpu.VMEM((DIM, BLK), jnp.float32),
            pltpu.VMEM((DIM, BLK), jnp.float32),
            pltpu.VMEM((DIM, BLK), jnp.float32),
            pltpu.VMEM((SCB, 2 * DIM), jnp.float32),
            pltpu.VMEM((SCB,), jnp.int32),
            pltpu.VMEM((18 * L,), jnp.int32),
            pltpu.VMEM((18 * L,), jnp.int32),
            pltpu.VMEM((18 * L,), jnp.int32),
            pltpu.SemaphoreType.DMA,
            pltpu.SemaphoreType.DMA,
            pltpu.SemaphoreType.DMA,
            pltpu.SemaphoreType.DMA,
            pltpu.SemaphoreType.DMA,
        ],
        compiler_params=pltpu.CompilerParams(needs_layout_passes=False),
    )
    stage = k1(hidx, tidx, ehT, eh_tail)

    k2 = pl.kernel(
        _k2_body,
        out_type=jax.ShapeDtypeStruct((B,), jnp.float32),
        mesh=mesh,
        scratch_types=[
            pltpu.VMEM((BPW,), jnp.int32),
            pltpu.VMEM((BPW,), jnp.int32),
            pltpu.VMEM((BPW,), jnp.int32),
            pltpu.VMEM((BPW,), jnp.int32),
        ] + [
            t
            for _ in range(2)
            for t in (pltpu.VMEM((128, 2 * DIM), jnp.float32),
                      pltpu.VMEM((128, 2 * DIM), jnp.float32),
                      pltpu.VMEM((128, 2 * DIM), jnp.float32),
                      pltpu.VMEM((128,), jnp.float32),
                      pltpu.VMEM((128,), jnp.float32))
        ] + [
            pltpu.VMEM((BPW,), jnp.float32),
            pltpu.SemaphoreType.DMA,
            pltpu.SemaphoreType.DMA,
        ],
        compiler_params=pltpu.CompilerParams(needs_layout_passes=False),
    )
    return k2(stage, hidx, tidx, ridx, rvh2, bias0, bias1)


# speedup vs baseline: 1.0310x; 1.0310x over previous
import functools
import jax
import jax.numpy as jnp
from jax import lax
from jax.experimental import pallas as pl
from jax.experimental.pallas import tpu as pltpu
from jax.experimental.pallas import tpu_sc as plsc

N_ENT = 1000000
N_REL = 1000
DIM = 64
B = 16384

NC = 2
NS = 16
L = 16
NW = NC * NS
BPW = B // NW
R = 2 * B
BLK = 128
NBLK_FULL = 7812
TAIL_BASE = NBLK_FULL * BLK
IT_FULL = 244
NORD = 246
DUMP = R
STAGE_ROWS = R + 128
SCB = 64
NRING = 4


def _k1_body(hidx, tidx, ehT, eh_tail, stage,
             idx_all, hits, horder, col0, col1, col2, col3, stage_b, slotbuf,
             cnt_v, off_v, cur_v,
             sem0, sem1, sem2, sem3, ssem):
    wid = lax.axis_index("s") * NC + lax.axis_index("c")
    lanes = lax.iota(jnp.int32, L)
    zero16 = jnp.full((L,), 0, jnp.int32)

    pltpu.sync_copy(hidx, idx_all.at[pl.ds(0, B)])

    def store1(ref, pos, val):
        plsc.store_scatter(ref, [zero16 + pos], zero16 + val,
                           mask=lanes == 0)

    def read1(ref, pos):
        return ref[pl.ds(pos, L)][0]

    def zcnt(i, carry):
        cnt_v[pl.ds(i * L, L)] = zero16
        return carry

    lax.fori_loop(0, (NORD + L - 1) // L + 1, zcnt, 0, unroll=False)

    def scan_for(slot_base):
        def scan(c, nh):
            e = idx_all[pl.ds(c * L, L)]
            slots = jnp.full((L,), slot_base + c * L, jnp.int32) + lanes
            m = lax.bitwise_and(lax.shift_right_logical(e, 7),
                                jnp.full((L,), NW - 1, jnp.int32)) == wid
            packed = lax.bitwise_or(
                lax.bitwise_or(
                    lax.shift_left(lax.shift_right_logical(e, 12),
                                   jnp.full((L,), 22, jnp.int32)),
                    lax.shift_left(
                        lax.bitwise_and(e, jnp.full((L,), BLK - 1,
                                                    jnp.int32)),
                        jnp.full((L,), 15, jnp.int32))),
                slots)
            plsc.store_compressed(hits.at[pl.ds(nh, L)], packed, mask=m)
            n = plsc.all_reduce_population_count(m)[0]
            return nh + n
        return scan

    nhits = lax.fori_loop(0, B // L, scan_for(0), 0, unroll=4)
    pltpu.sync_copy(tidx, idx_all.at[pl.ds(0, B)])
    nhits = lax.fori_loop(0, B // L, scan_for(B), nhits, unroll=4)

    def count(j, carry):
        o = lax.shift_right_logical(read1(hits, j), 22)
        store1(cnt_v, o, read1(cnt_v, o) + 1)
        return carry

    lax.fori_loop(0, nhits, count, 0, unroll=False)

    def prefix(i, s):
        store1(off_v, i, s)
        store1(cur_v, i, s)
        return s + read1(cnt_v, i)

    total = lax.fori_loop(0, NORD, prefix, 0, unroll=False)
    store1(off_v, NORD, total)

    def place(j, carry):
        v = read1(hits, j)
        o = lax.shift_right_logical(v, 22)
        p = read1(cur_v, o)
        store1(horder, p, v)
        store1(cur_v, o, p + 1)
        return carry

    lax.fori_loop(0, nhits, place, 0, unroll=False)

    def prefill(i, carry):
        slotbuf[pl.ds(i * L, L)] = jnp.full((L,), DUMP, jnp.int32)
        return carry

    lax.fori_loop(0, SCB // L, prefill, 0, unroll=False)

    def process_block(i, col, cnt):
        lo = read1(off_v, i)
        hi = read1(off_v, i + 1)

        def extract(j, cnt3):
            flushing = cnt3 == SCB

            @pl.when(flushing)
            def _():
                pltpu.async_copy(stage_b, stage.at[slotbuf], ssem).wait()
                lax.fori_loop(0, SCB // L, prefill, 0, unroll=False)

            cnt3 = jnp.where(flushing, 0, cnt3)
            v = read1(horder, j)
            slot = lax.bitwise_and(v, (1 << 15) - 1)
            el = lax.bitwise_and(lax.shift_right_logical(v, 15), BLK - 1)
            elv = zero16 + el
            rowv = zero16 + cnt3
            for k in range(DIM // L):
                dv = jnp.full((L,), k * L, jnp.int32) + lanes
                vv = plsc.load_gather(col, [dv, elv])
                plsc.store_scatter(stage_b, [rowv, dv], vv)
            store1(slotbuf, cnt3, slot)
            return cnt3 + 1

        return lax.fori_loop(lo, hi, extract, cnt, unroll=False)

    def pbase(i):
        return jnp.minimum((wid + NW * i) * BLK, (NBLK_FULL - 1) * BLK)

    cols = [col0, col1, col2, col3]
    sems = [sem0, sem1, sem2, sem3]
    for k in range(NRING):
        pltpu.async_copy(ehT.at[:, pl.ds(pbase(k), BLK)], cols[k], sems[k])

    def ring(g, cnt):
        for k in range(NRING):
            i = NRING * g + k
            pltpu.make_async_copy(ehT.at[:, pl.ds(pbase(i), BLK)],
                                  cols[k], sems[k]).wait()
            cnt = process_block(i, cols[k], cnt)
            pltpu.async_copy(ehT.at[:, pl.ds(pbase(i + NRING), BLK)],
                             cols[k], sems[k])
        return cnt

    cnt = lax.fori_loop(0, IT_FULL // NRING, ring, 0, unroll=False)

    for k in range(NRING):
        pltpu.make_async_copy(ehT.at[:, pl.ds(pbase(IT_FULL + k), BLK)],
                              cols[k], sems[k]).wait()

    @pl.when(wid < 4)
    def _():
        pltpu.sync_copy(ehT.at[:, pl.ds((wid + NW * IT_FULL) * BLK, BLK)],
                        col0)

    @pl.when(wid == 4)
    def _():
        pltpu.sync_copy(eh_tail, col0)

    cnt = process_block(IT_FULL, col0, cnt)

    @pl.when(cnt > 0)
    def _():
        pltpu.async_copy(stage_b, stage.at[slotbuf], ssem).wait()


def _k2_body(stage, hidx, tidx, ridx, rvh2, b0, b1, out,
             rel_v, rpair_v, hi_v, ti_v,
             h_r0, t_r0, rv_r0, b0_v0, b1_v0,
             h_r1, t_r1, rv_r1, b0_v1, b1_v1,
             out_v, semA, semB):
    wid = lax.axis_index("s") * NC + lax.axis_index("c")
    base = wid * BPW
    lanes = lax.iota(jnp.int32, L)

    pltpu.sync_copy(ridx.at[pl.ds(base, BPW)], rel_v)
    pltpu.sync_copy(hidx.at[pl.ds(base, BPW)], hi_v)
    pltpu.sync_copy(tidx.at[pl.ds(base, BPW)], ti_v)

    def pairs(i, carry):
        s = pl.ds(i * L, L)
        rpair_v[s] = lax.shift_right_logical(rel_v[s], 1)
        return carry

    lax.fori_loop(0, BPW // L, pairs, 0, unroll=False)

    KB = 128
    NBAT = BPW // KB
    sets = [(h_r0, t_r0, rv_r0, b0_v0, b1_v0, semA),
            (h_r1, t_r1, rv_r1, b0_v1, b1_v1, semB)]

    def fire(bi, st):
        h_r, t_r, rv_r, b0_v, b1_v, sem = st
        s0 = base + bi * KB
        return [
            pltpu.async_copy(stage.at[pl.ds(s0, KB), :], h_r, sem),
            pltpu.async_copy(stage.at[pl.ds(B + s0, KB), :], t_r, sem),
            pltpu.async_copy(rvh2.at[rpair_v.at[pl.ds(bi * KB, KB)]],
                             rv_r, sem),
            pltpu.async_copy(b0.at[hi_v.at[pl.ds(bi * KB, KB)]], b0_v, sem),
            pltpu.async_copy(b1.at[ti_v.at[pl.ds(bi * KB, KB)]], b1_v, sem),
        ]

    def compute(bi, st):
        h_r, t_r, rv_r, b0_v, b1_v, _ = st

        def group(g, carry2):
            req = jnp.full((L,), g * L, jnp.int32) + lanes
            rh = lax.bitwise_and(
                rel_v[pl.ds(bi * KB + g * L, L)],
                jnp.full((L,), 1, jnp.int32)) * DIM
            acc = jnp.zeros((L,), jnp.float32)
            for d in range(DIM):
                col = jnp.full((L,), d, jnp.int32)
                hv = plsc.load_gather(h_r, [req, col])
                tv = plsc.load_gather(t_r, [req, col])
                rv = plsc.load_gather(rv_r, [req, rh + col])
                diff = hv - tv - rv
                acc = acc + diff * diff
            gs = pl.ds(g * L, L)
            out_v[pl.ds(bi * KB + g * L, L)] = b0_v[gs] + b1_v[gs] - acc
            return carry2

        lax.fori_loop(0, KB // L, group, 0, unroll=False)

    ring = len(sets)
    pending = fire(0, sets[0])
    for bi in range(NBAT):
        st = sets[bi % ring]
        for c in pending:
            c.wait()
        if bi + 1 < NBAT:
            nxt = fire(bi + 1, sets[(bi + 1) % ring])
        else:
            nxt = []
        compute(bi, st)
        pending = nxt

    pltpu.sync_copy(out_v, out.at[pl.ds(base, BPW)])


@functools.partial(jax.jit, static_argnames=())
def kernel(head_idx, rel1_idx, tail_idx, rel2_idx, Eh, rvh, bias0, bias1):
    del rel2_idx
    hidx = head_idx.astype(jnp.int32)
    tidx = tail_idx.astype(jnp.int32)
    ridx = rel1_idx.astype(jnp.int32)
    ehT = Eh.T
    eh_tail = jnp.pad(Eh[TAIL_BASE:, :].T, ((0, 0), (0, DIM)))
    rvh2 = rvh.reshape(N_REL // 2, 2 * DIM)
    mesh = plsc.VectorSubcoreMesh(core_axis_name="c", subcore_axis_name="s")

    k1 = pl.kernel(
        _k1_body,
        out_type=jax.ShapeDtypeStruct((STAGE_ROWS, 2 * DIM), jnp.float32),
        mesh=mesh,
        scratch_types=[
            pltpu.VMEM((B + L,), jnp.int32),
            pltpu.VMEM((R + L,), jnp.int32),
            pltpu.VMEM((R + L,), jnp.int32),
            pltpu.VMEM((DIM, BLK), jnp.float32),
            pltpu.VMEM((DIM, BLK), jnp.float32),
            pltpu.VMEM((DIM, BLK), jnp.float32),
            pltpu.VMEM((DIM, BLK), jnp.float32),
            pltpu.VMEM((SCB, 2 * DIM), jnp.float32),
            pltpu.VMEM((SCB,), jnp.int32),
            pltpu.VMEM((18 * L,), jnp.int32),
            pltpu.VMEM((18 * L,), jnp.int32),
            pltpu.VMEM((18 * L,), jnp.int32),
            pltpu.SemaphoreType.DMA,
            pltpu.SemaphoreType.DMA,
            pltpu.SemaphoreType.DMA,
            pltpu.SemaphoreType.DMA,
            pltpu.SemaphoreType.DMA,
        ],
        compiler_params=pltpu.CompilerParams(needs_layout_passes=False),
    )
    stage = k1(hidx, tidx, ehT, eh_tail)

    k2 = pl.kernel(
        _k2_body,
        out_type=jax.ShapeDtypeStruct((B,), jnp.float32),
        mesh=mesh,
        scratch_types=[
            pltpu.VMEM((BPW,), jnp.int32),
            pltpu.VMEM((BPW,), jnp.int32),
            pltpu.VMEM((BPW,), jnp.int32),
            pltpu.VMEM((BPW,), jnp.int32),
        ] + [
            t
            for _ in range(2)
            for t in (pltpu.VMEM((128, 2 * DIM), jnp.float32),
                      pltpu.VMEM((128, 2 * DIM), jnp.float32),
                      pltpu.VMEM((128, 2 * DIM), jnp.float32),
                      pltpu.VMEM((128,), jnp.float32),
                      pltpu.VMEM((128,), jnp.float32))
        ] + [
            pltpu.VMEM((BPW,), jnp.float32),
            pltpu.SemaphoreType.DMA,
            pltpu.SemaphoreType.DMA,
        ],
        compiler_params=pltpu.CompilerParams(needs_layout_passes=False),
    )
    return k2(stage, hidx, tidx, ridx, rvh2, bias0, bias1)
